# baseline (speedup 1.0000x reference)
import functools

import jax
import jax.numpy as jnp
from jax import lax
from jax.experimental import pallas as pl
from jax.experimental.pallas import tpu as pltpu
from jax.experimental.pallas import tpu_sc as plsc

NC = 2
SC0_FRAC = 0.75
NS = 16
NW = NC * NS
CH = 64


def _sc_agg_body(K0, K1, ROWS_PT, tbl_hbm, cidx_hbm, dst_hbm,
                 acc_hbm, slab, idx_dA, hrA, idx_dB, hrB,
                 acc_sh, semA, semB, sem_scA, sem_scB, semI):
    cid = lax.axis_index("c")
    sid = lax.axis_index("s")
    cbase = jnp.where(cid == 0, sid * K0, NS * K0 + sid * K1)
    nchunk = jnp.where(cid == 0, K0, K1)

    @pl.when(cid == 0)
    def _():
        pltpu.async_copy(cidx_hbm.at[pl.ds(sid * K0, K0)], slab, semI)

    @pl.when(cid == 1)
    def _():
        pltpu.async_copy(cidx_hbm.at[pl.ds(NS * K0 + sid * K1, K1)],
                         slab.at[pl.ds(0, K1)], semI)

    zeros16 = jnp.zeros((16,), jnp.float32)

    def zrow(r, c):
        for j in range(8):
            hrA[r, pl.ds(j * 16, 16)] = zeros16
        return c

    lax.fori_loop(0, 2 * CH, zrow, 0)

    def zcopy(k, c):
        pltpu.sync_copy(hrA, acc_sh.at[pl.ds(sid * ROWS_PT + k * 2 * CH,
                                             2 * CH)])
        return c

    lax.fori_loop(0, ROWS_PT // (2 * CH), zcopy, 0)
    plsc.subcore_barrier()

    @pl.when(cid == 0)
    def _():
        pltpu.make_async_copy(cidx_hbm.at[pl.ds(sid * K0, K0)], slab,
                              semI).wait()

    @pl.when(cid == 1)
    def _():
        pltpu.make_async_copy(cidx_hbm.at[pl.ds(NS * K0 + sid * K1, K1)],
                              slab.at[pl.ds(0, K1)], semI).wait()

    dbase = cbase * CH

    def stage(chunk_i, idx_d, hr, sem, sem_sc, scatter_pending):
        if scatter_pending:
            pltpu.make_async_copy(hr.at[pl.ds(CH, CH)], acc_sh.at[idx_d],
                                  sem_sc).wait()
        pltpu.async_copy(dst_hbm.at[pl.ds(dbase + chunk_i * CH, CH)], idx_d,
                         sem)
        pltpu.async_copy(tbl_hbm.at[slab.at[chunk_i]], hr, sem)

    def drain(idx_d, hr, sem):
        pltpu.make_async_copy(dst_hbm.at[pl.ds(dbase, CH)], idx_d, sem).wait()
        pltpu.make_async_copy(tbl_hbm.at[slab.at[0]], hr, sem).wait()

    def compute(idx_d, hr, sem_sc):
        def sub_rows(r):
            for rr in range(4):
                for j in range(8):
                    sl = pl.ds(j * 16, 16)
                    hr[CH + r + rr, sl] = hr[r + rr, sl] - hr[CH + r + rr, sl]

        plsc.parallel_loop(0, CH, 4)(sub_rows)
        pltpu.async_copy(hr.at[pl.ds(CH, CH)], acc_sh.at[idx_d], sem_sc,
                         add=True)

    stage(0, idx_dA, hrA, semA, semA, False)

    def pipe_body(j, c, first):
        c0 = 2 * j
        stage(c0 + 1, idx_dB, hrB, semB, sem_scB, not first)
        drain(idx_dA, hrA, semA)
        compute(idx_dA, hrA, sem_scA)

        @pl.when(c0 + 2 < nchunk)
        def _():
            stage(c0 + 2, idx_dA, hrA, semA, sem_scA, True)

        drain(idx_dB, hrB, semB)
        compute(idx_dB, hrB, sem_scB)
        return c

    pipe_body(0, 0, True)
    lax.fori_loop(1, nchunk // 2, lambda j, c: pipe_body(j, c, False), 0)
    pltpu.make_async_copy(hrA.at[pl.ds(CH, CH)], acc_sh.at[idx_dA],
                          sem_scA).wait()
    pltpu.make_async_copy(hrB.at[pl.ds(CH, CH)], acc_sh.at[idx_dB],
                          sem_scB).wait()
    plsc.subcore_barrier()

    pltpu.sync_copy(acc_sh.at[pl.ds(sid * ROWS_PT, ROWS_PT)],
                    acc_hbm.at[cid, pl.ds(sid * ROWS_PT, ROWS_PT)])


def _tc_body(x_ref, a0_ref, a1_ref, rel_ref, ws_ref, wf_ref, wr_ref, b_ref,
             o_ref, rout_ref):
    a = a0_ref[0] + a1_ref[0]
    acc = jnp.dot(x_ref[...], ws_ref[...], preferred_element_type=jnp.float32)
    acc = acc + jnp.dot(a, wf_ref[...], preferred_element_type=jnp.float32)
    o_ref[...] = acc + b_ref[...]

    @pl.when(pl.program_id(0) == 0)
    def _():
        rout_ref[...] = jnp.dot(rel_ref[...], wr_ref[...],
                                preferred_element_type=jnp.float32)


def kernel(node_feat, rel_emb, edge_index, edge_type, W_self, W_forward,
           W_rel, bias):
    N, D = node_feat.shape
    R = rel_emb.shape[0]
    OUT = W_self.shape[0]
    E = edge_index.shape[1]
    EF = E // 2

    EPW = -(-EF // (NW * 2 * CH)) * (2 * CH)
    NP = -(-(N + 1) // (NS * 2 * CH)) * (NS * 2 * CH)
    ROWS_PT = NP // NS
    pad = NW * EPW - EF

    PAIR = 2 * EPW
    EPW0 = int(round(PAIR * SC0_FRAC / (8 * CH))) * (8 * CH)
    EPW0 = min(EPW0, PAIR - 8 * CH)
    EPW1 = PAIR - EPW0
    K0, K1 = EPW0 // CH, EPW1 // CH
    TCHUNK = NS * (K0 + K1)
    dummy_dst = N + jnp.arange(pad, dtype=jnp.int32) % (NP - N)
    pads = jnp.stack([jnp.zeros((pad,), jnp.int32),
                      jnp.full((pad,), N, jnp.int32), dummy_dst])
    fwd = jnp.stack([edge_index[0, :EF], edge_type[:EF] + N,
                     edge_index[1, :EF]])
    edges = jnp.concatenate([fwd, pads], axis=1)
    dst = edges[2]
    tbl = jnp.concatenate([node_feat, rel_emb], axis=0)
    cidx = jnp.concatenate([edges[0].reshape(TCHUNK, CH),
                            edges[1].reshape(TCHUNK, CH)], axis=1)

    mesh = plsc.VectorSubcoreMesh(core_axis_name="c", subcore_axis_name="s",
                                  num_cores=NC, num_subcores=NS)
    sc_agg = pl.kernel(
        functools.partial(_sc_agg_body, K0, K1, ROWS_PT),
        out_type=jax.ShapeDtypeStruct((NC, NP, D), jnp.float32),
        mesh=mesh,
        scratch_types=[
            pltpu.VMEM((K0, 2 * CH), jnp.int32),
            pltpu.VMEM((CH,), jnp.int32),
            pltpu.VMEM((2 * CH, D), jnp.float32),
            pltpu.VMEM((CH,), jnp.int32),
            pltpu.VMEM((2 * CH, D), jnp.float32),
            pltpu.VMEM_SHARED((NP, D), jnp.float32),
            pltpu.SemaphoreType.DMA,
            pltpu.SemaphoreType.DMA,
            pltpu.SemaphoreType.DMA,
            pltpu.SemaphoreType.DMA,
            pltpu.SemaphoreType.DMA,
        ],
    )
    acc = sc_agg(tbl, cidx, dst)

    BN = 2000
    grid = N // BN
    out, rel_out = pl.pallas_call(
        _tc_body,
        grid=(grid,),
        in_specs=[
            pl.BlockSpec((BN, D), lambda i: (i, 0)),
            pl.BlockSpec((1, BN, D), lambda i: (0, i, 0)),
            pl.BlockSpec((1, BN, D), lambda i: (1, i, 0)),
            pl.BlockSpec((R, D), lambda i: (0, 0)),
            pl.BlockSpec((D, OUT), lambda i: (0, 0)),
            pl.BlockSpec((D, OUT), lambda i: (0, 0)),
            pl.BlockSpec((D, OUT), lambda i: (0, 0)),
            pl.BlockSpec((1, OUT), lambda i: (0, 0)),
        ],
        out_specs=[
            pl.BlockSpec((BN, OUT), lambda i: (i, 0)),
            pl.BlockSpec((R, OUT), lambda i: (0, 0)),
        ],
        out_shape=[
            jax.ShapeDtypeStruct((N, OUT), jnp.float32),
            jax.ShapeDtypeStruct((R, OUT), jnp.float32),
        ],
    )(node_feat, acc, acc, rel_emb, W_self.T, W_forward.T, W_rel.T,
      bias.reshape(1, OUT))
    return (out, rel_out)


# speedup vs baseline: 1.1628x; 1.1628x over previous
import functools

import jax
import jax.numpy as jnp
from jax import lax
from jax.experimental import pallas as pl
from jax.experimental.pallas import tpu as pltpu
from jax.experimental.pallas import tpu_sc as plsc

NC = 2
SC0_FRAC = 0.75
NS = 16
NW = NC * NS
CH = 64


def _sc_agg_body(K0, K1, ROWS_PT, tbl_hbm, cidx_hbm, dst_hbm,
                 acc_hbm, slab, idx_dA, hrA, idx_dB, hrB,
                 acc_sh, semA, semB, sem_scA, sem_scB, semI):
    cid = lax.axis_index("c")
    sid = lax.axis_index("s")
    cbase = jnp.where(cid == 0, sid * K0, NS * K0 + sid * K1)
    nchunk = jnp.where(cid == 0, K0, K1)

    @pl.when(cid == 0)
    def _():
        pltpu.async_copy(cidx_hbm.at[pl.ds(sid * K0, K0)], slab, semI)

    @pl.when(cid == 1)
    def _():
        pltpu.async_copy(cidx_hbm.at[pl.ds(NS * K0 + sid * K1, K1)],
                         slab.at[pl.ds(0, K1)], semI)

    zeros16 = jnp.zeros((16,), jnp.float32)

    def zrow(r, c):
        for j in range(8):
            hrA[r, pl.ds(j * 16, 16)] = zeros16
        return c

    lax.fori_loop(0, 2 * CH, zrow, 0)

    def zcopy(k, c):
        pltpu.sync_copy(hrA, acc_sh.at[pl.ds(sid * ROWS_PT + k * 2 * CH,
                                             2 * CH)])
        return c

    lax.fori_loop(0, ROWS_PT // (2 * CH), zcopy, 0)
    plsc.subcore_barrier()

    @pl.when(cid == 0)
    def _():
        pltpu.make_async_copy(cidx_hbm.at[pl.ds(sid * K0, K0)], slab,
                              semI).wait()

    @pl.when(cid == 1)
    def _():
        pltpu.make_async_copy(cidx_hbm.at[pl.ds(NS * K0 + sid * K1, K1)],
                              slab.at[pl.ds(0, K1)], semI).wait()

    dbase = cbase * CH

    def stage(chunk_i, idx_d, hr, sem, sem_sc, scatter_pending):
        if scatter_pending:
            pltpu.make_async_copy(hr.at[pl.ds(CH, CH)], acc_sh.at[idx_d],
                                  sem_sc).wait()
        pltpu.async_copy(dst_hbm.at[pl.ds(dbase + chunk_i * CH, CH)], idx_d,
                         sem)
        pltpu.async_copy(tbl_hbm.at[slab.at[chunk_i]], hr, sem)

    def drain(idx_d, hr, sem):
        pltpu.make_async_copy(dst_hbm.at[pl.ds(dbase, CH)], idx_d, sem).wait()
        pltpu.make_async_copy(tbl_hbm.at[slab.at[0]], hr, sem).wait()

    def compute(idx_d, hr, sem_sc):
        def sub_rows(r):
            for rr in range(4):
                for j in range(8):
                    sl = pl.ds(j * 16, 16)
                    hr[CH + r + rr, sl] = hr[r + rr, sl] - hr[CH + r + rr, sl]

        plsc.parallel_loop(0, CH, 4)(sub_rows)
        pltpu.async_copy(hr.at[pl.ds(CH, CH)], acc_sh.at[idx_d], sem_sc,
                         add=True)

    stage(0, idx_dA, hrA, semA, semA, False)

    def pipe_body(j, c, first):
        c0 = 2 * j
        stage(c0 + 1, idx_dB, hrB, semB, sem_scB, not first)
        drain(idx_dA, hrA, semA)
        compute(idx_dA, hrA, sem_scA)

        @pl.when(c0 + 2 < nchunk)
        def _():
            stage(c0 + 2, idx_dA, hrA, semA, sem_scA, True)

        drain(idx_dB, hrB, semB)
        compute(idx_dB, hrB, sem_scB)
        return c

    pipe_body(0, 0, True)
    lax.fori_loop(1, nchunk // 2, lambda j, c: pipe_body(j, c, False), 0)
    pltpu.make_async_copy(hrA.at[pl.ds(CH, CH)], acc_sh.at[idx_dA],
                          sem_scA).wait()
    pltpu.make_async_copy(hrB.at[pl.ds(CH, CH)], acc_sh.at[idx_dB],
                          sem_scB).wait()
    plsc.subcore_barrier()

    pltpu.sync_copy(acc_sh.at[pl.ds(sid * ROWS_PT, ROWS_PT)],
                    acc_hbm.at[cid, pl.ds(sid * ROWS_PT, ROWS_PT)])


def _tc_body(x_ref, a0_ref, a1_ref, rel_ref, ws_ref, wf_ref, wr_ref, b_ref,
             o_ref, rout_ref):
    a = a0_ref[0] + a1_ref[0]
    acc = jnp.dot(x_ref[...], ws_ref[...], preferred_element_type=jnp.float32)
    acc = acc + jnp.dot(a, wf_ref[...], preferred_element_type=jnp.float32)
    o_ref[...] = acc + b_ref[...]

    @pl.when(pl.program_id(0) == 0)
    def _():
        rout_ref[...] = jnp.dot(rel_ref[...], wr_ref[...],
                                preferred_element_type=jnp.float32)


def kernel(node_feat, rel_emb, edge_index, edge_type, W_self, W_forward,
           W_rel, bias):
    N, D = node_feat.shape
    R = rel_emb.shape[0]
    OUT = W_self.shape[0]
    E = edge_index.shape[1]
    EF = E // 2

    EPW = -(-EF // (NW * 2 * CH)) * (2 * CH)
    NP = -(-(N + 1) // (NS * 2 * CH)) * (NS * 2 * CH)
    ROWS_PT = NP // NS
    pad = NW * EPW - EF

    PAIR = 2 * EPW
    EPW0 = int(round(PAIR * SC0_FRAC / (8 * CH))) * (8 * CH)
    EPW0 = min(EPW0, PAIR - 8 * CH)
    EPW1 = PAIR - EPW0
    K0, K1 = EPW0 // CH, EPW1 // CH
    TCHUNK = NS * (K0 + K1)
    src = jnp.concatenate([edge_index[0, :EF], jnp.zeros((pad,), jnp.int32)])
    dummy_dst = N + jnp.arange(pad, dtype=jnp.int32) % (NP - N)
    dst = jnp.concatenate([edge_index[1, :EF], dummy_dst])
    et = jnp.concatenate([edge_type[:EF], jnp.zeros((pad,), jnp.int32)])
    tbl = jnp.concatenate([node_feat, rel_emb], axis=0)
    cidx = jnp.concatenate([src.reshape(TCHUNK, CH),
                            (et + N).reshape(TCHUNK, CH)], axis=1)

    mesh = plsc.VectorSubcoreMesh(core_axis_name="c", subcore_axis_name="s",
                                  num_cores=NC, num_subcores=NS)
    sc_agg = pl.kernel(
        functools.partial(_sc_agg_body, K0, K1, ROWS_PT),
        out_type=jax.ShapeDtypeStruct((NC, NP, D), jnp.float32),
        mesh=mesh,
        scratch_types=[
            pltpu.VMEM((K0, 2 * CH), jnp.int32),
            pltpu.VMEM((CH,), jnp.int32),
            pltpu.VMEM((2 * CH, D), jnp.float32),
            pltpu.VMEM((CH,), jnp.int32),
            pltpu.VMEM((2 * CH, D), jnp.float32),
            pltpu.VMEM_SHARED((NP, D), jnp.float32),
            pltpu.SemaphoreType.DMA,
            pltpu.SemaphoreType.DMA,
            pltpu.SemaphoreType.DMA,
            pltpu.SemaphoreType.DMA,
            pltpu.SemaphoreType.DMA,
        ],
    )
    acc = sc_agg(tbl, cidx, dst)

    BN = 1000
    grid = N // BN
    out, rel_out = pl.pallas_call(
        _tc_body,
        grid=(grid,),
        in_specs=[
            pl.BlockSpec((BN, D), lambda i: (i, 0)),
            pl.BlockSpec((1, BN, D), lambda i: (0, i, 0)),
            pl.BlockSpec((1, BN, D), lambda i: (1, i, 0)),
            pl.BlockSpec((R, D), lambda i: (0, 0)),
            pl.BlockSpec((D, OUT), lambda i: (0, 0)),
            pl.BlockSpec((D, OUT), lambda i: (0, 0)),
            pl.BlockSpec((D, OUT), lambda i: (0, 0)),
            pl.BlockSpec((1, OUT), lambda i: (0, 0)),
        ],
        out_specs=[
            pl.BlockSpec((BN, OUT), lambda i: (i, 0)),
            pl.BlockSpec((R, OUT), lambda i: (0, 0)),
        ],
        out_shape=[
            jax.ShapeDtypeStruct((N, OUT), jnp.float32),
            jax.ShapeDtypeStruct((R, OUT), jnp.float32),
        ],
    )(node_feat, acc, acc, rel_emb, W_self.T, W_forward.T, W_rel.T,
      bias.reshape(1, OUT))
    return (out, rel_out)
